# baseline (speedup 1.0000x reference)
import functools

import jax
import jax.numpy as jnp
from jax import lax
from jax.experimental import pallas as pl
from jax.experimental.pallas import tpu as pltpu
from jax.experimental.pallas import tpu_sc as plsc

VOCAB = 1000000
SEG = 64
DIM = 64
B, S = 1024, 200
N = B * S
NC, NS, L = 2, 16, 16
NW = NC * NS
TOK_PER_W = N // NW
CHUNK = 128
NCHUNK = TOK_PER_W // CHUNK
CW = 2 * DIM


def _sc_kernel(word_hbm, cidx_hbm, w_hbm, c_hbm, out_hbm,
               widx_all, cidx_all,
               crows0, crows1, crows2, obuf0, obuf1, obuf2,
               sem_c0, sem_c1, sem_c2, sem_g0, sem_g1, sem_g2,
               sem_o0, sem_o1, sem_o2):
    wid = lax.axis_index("s") * NC + lax.axis_index("c")
    base = wid * TOK_PER_W

    crows = (crows0, crows1, crows2)
    obuf = (obuf0, obuf1, obuf2)
    sem_c = (sem_c0, sem_c1, sem_c2)
    sem_g = (sem_g0, sem_g1, sem_g2)
    sem_o = (sem_o0, sem_o1, sem_o2)

    pltpu.sync_copy(word_hbm.at[pl.ds(base, TOK_PER_W)], widx_all)
    pltpu.sync_copy(cidx_hbm.at[pl.ds(base, TOK_PER_W)], cidx_all)

    def prefetch(k, b):
        pltpu.async_copy(c_hbm.at[cidx_all.at[pl.ds(k * CHUNK, CHUNK)]],
                         crows[b], sem_c[b])

        def fire(t16, c2):
            v = widx_all[pl.ds(k * CHUNK + t16 * L, L)]
            for j in range(L):
                pltpu.async_copy(w_hbm.at[v[j]], obuf[b].at[t16 * L + j],
                                 sem_g[b])
            return c2
        lax.fori_loop(0, CHUNK // L, fire, 0, unroll=2)

    def drain_gather(b):
        pltpu.make_async_copy(w_hbm.at[pl.ds(0, CHUNK)], obuf[b],
                              sem_g[b]).wait()
        pltpu.make_async_copy(c_hbm.at[pl.ds(0, CHUNK)], crows[b],
                              sem_c[b]).wait()

    def compute(b):
        def tok_body(t, c2):
            for c in range(DIM // L):
                sl = pl.ds(c * L, L)
                plsc.addupdate(obuf[b].at[t, sl], crows[b][t, sl])
            return c2
        lax.fori_loop(0, CHUNK, tok_body, 0, unroll=8)

    def store(k, b):
        pltpu.async_copy(obuf[b], out_hbm.at[pl.ds(base + k * CHUNK, CHUNK)],
                         sem_o[b])

    def wait_store(b):
        pltpu.make_async_copy(out_hbm.at[pl.ds(0, CHUNK)], obuf[b],
                              sem_o[b]).wait()

    prefetch(0, 0)

    def chunk_body(k, carry):
        b = lax.rem(k, 3)
        for p in range(3):
            pn = (p + 1) % 3

            @pl.when((b == p) & (k + 1 < NCHUNK))
            def _():
                @pl.when(k >= 2)
                def _():
                    wait_store(pn)
                prefetch(k + 1, pn)

        for p in range(3):
            @pl.when(b == p)
            def _():
                drain_gather(p)
                compute(p)
                store(k, p)
        return carry

    lax.fori_loop(0, NCHUNK, chunk_body, 0)
    wait_store(0)
    wait_store(1)
    wait_store(2)


@jax.jit
def _run(word_flat, cidx_flat, W, Cpad):
    mesh = plsc.VectorSubcoreMesh(core_axis_name="c", subcore_axis_name="s")
    f = functools.partial(
        pl.kernel,
        mesh=mesh,
        compiler_params=pltpu.CompilerParams(use_tc_tiling_on_sc=True),
        out_type=jax.ShapeDtypeStruct((N, DIM), jnp.float32),
        scratch_types=[
            pltpu.VMEM((TOK_PER_W,), jnp.int32),
            pltpu.VMEM((TOK_PER_W,), jnp.int32),
            pltpu.VMEM((CHUNK, CW), jnp.float32),
            pltpu.VMEM((CHUNK, CW), jnp.float32),
            pltpu.VMEM((CHUNK, CW), jnp.float32),
            pltpu.VMEM((CHUNK, DIM), jnp.float32),
            pltpu.VMEM((CHUNK, DIM), jnp.float32),
            pltpu.VMEM((CHUNK, DIM), jnp.float32),
            pltpu.SemaphoreType.DMA,
            pltpu.SemaphoreType.DMA,
            pltpu.SemaphoreType.DMA,
            pltpu.SemaphoreType.DMA,
            pltpu.SemaphoreType.DMA,
            pltpu.SemaphoreType.DMA,
            pltpu.SemaphoreType.DMA,
            pltpu.SemaphoreType.DMA,
            pltpu.SemaphoreType.DMA,
        ],
    )(_sc_kernel)
    return f(word_flat, cidx_flat, W, Cpad)


def kernel(word_ids, mask_ids, W, P, Sg):
    word_flat = word_ids.reshape(-1).astype(jnp.int32)
    mask_flat = mask_ids.reshape(-1).astype(jnp.int32)
    pos_flat = jnp.broadcast_to(jnp.arange(S, dtype=jnp.int32),
                                (B, S)).reshape(-1)
    cidx_flat = mask_flat * S + pos_flat
    C = (Sg[:, None, :] + P[None, :S, :]).reshape(SEG * S, DIM)
    Cpad = jnp.pad(C, ((0, 0), (0, CW - DIM)))
    out = _run(word_flat, cidx_flat, W, Cpad)
    return out.reshape(B, S, DIM)


# speedup vs baseline: 1.0054x; 1.0054x over previous
import functools

import jax
import jax.numpy as jnp
from jax import lax
from jax.experimental import pallas as pl
from jax.experimental.pallas import tpu as pltpu
from jax.experimental.pallas import tpu_sc as plsc

VOCAB = 1000000
SEG = 64
DIM = 64
B, S = 1024, 200
N = B * S
NC, NS, L = 2, 16, 16
NW = NC * NS
TOK_PER_W = N // NW
CHUNK = 128
NCHUNK = TOK_PER_W // CHUNK
CW = 2 * DIM


def _sc_kernel(word_hbm, cidx_hbm, w_hbm, c_hbm, out_hbm,
               widx_all, cidx_all,
               crows0, crows1, crows2, obuf0, obuf1, obuf2,
               sem_c0, sem_c1, sem_c2, sem_g0, sem_g1, sem_g2,
               sem_o0, sem_o1, sem_o2):
    wid = lax.axis_index("s") * NC + lax.axis_index("c")
    base = wid * TOK_PER_W

    crows = (crows0, crows1, crows2)
    obuf = (obuf0, obuf1, obuf2)
    sem_c = (sem_c0, sem_c1, sem_c2)
    sem_g = (sem_g0, sem_g1, sem_g2)
    sem_o = (sem_o0, sem_o1, sem_o2)

    pltpu.sync_copy(word_hbm.at[pl.ds(base, TOK_PER_W)], widx_all)
    pltpu.sync_copy(cidx_hbm.at[pl.ds(base, TOK_PER_W)], cidx_all)

    def prefetch(k, b):
        pltpu.async_copy(c_hbm.at[cidx_all.at[pl.ds(k * CHUNK, CHUNK)]],
                         crows[b], sem_c[b])

        def fire(t16, c2):
            v = widx_all[pl.ds(k * CHUNK + t16 * L, L)]
            for j in range(L):
                pltpu.async_copy(w_hbm.at[v[j]], obuf[b].at[t16 * L + j],
                                 sem_g[b])
            return c2
        lax.fori_loop(0, CHUNK // L, fire, 0)

    def drain_gather(b):
        pltpu.make_async_copy(w_hbm.at[pl.ds(0, CHUNK)], obuf[b],
                              sem_g[b]).wait()
        pltpu.make_async_copy(c_hbm.at[pl.ds(0, CHUNK)], crows[b],
                              sem_c[b]).wait()

    def compute(b):
        def tok_body(t, c2):
            for c in range(DIM // L):
                sl = pl.ds(c * L, L)
                plsc.addupdate(obuf[b].at[t, sl], crows[b][t, sl])
            return c2
        lax.fori_loop(0, CHUNK, tok_body, 0, unroll=4)

    def store(k, b):
        pltpu.async_copy(obuf[b], out_hbm.at[pl.ds(base + k * CHUNK, CHUNK)],
                         sem_o[b])

    def wait_store(b):
        pltpu.make_async_copy(out_hbm.at[pl.ds(0, CHUNK)], obuf[b],
                              sem_o[b]).wait()

    prefetch(0, 0)

    def chunk_body(k, carry):
        b = lax.rem(k, 3)
        for p in range(3):
            pn = (p + 1) % 3

            @pl.when((b == p) & (k + 1 < NCHUNK))
            def _():
                @pl.when(k >= 2)
                def _():
                    wait_store(pn)
                prefetch(k + 1, pn)

        for p in range(3):
            @pl.when(b == p)
            def _():
                drain_gather(p)
                compute(p)
                store(k, p)
        return carry

    lax.fori_loop(0, NCHUNK, chunk_body, 0)
    wait_store(0)
    wait_store(1)
    wait_store(2)


@jax.jit
def _run(word_flat, cidx_flat, W, Cpad):
    mesh = plsc.VectorSubcoreMesh(core_axis_name="c", subcore_axis_name="s")
    f = functools.partial(
        pl.kernel,
        mesh=mesh,
        compiler_params=pltpu.CompilerParams(use_tc_tiling_on_sc=True),
        out_type=jax.ShapeDtypeStruct((N, DIM), jnp.float32),
        scratch_types=[
            pltpu.VMEM((TOK_PER_W,), jnp.int32),
            pltpu.VMEM((TOK_PER_W,), jnp.int32),
            pltpu.VMEM((CHUNK, CW), jnp.float32),
            pltpu.VMEM((CHUNK, CW), jnp.float32),
            pltpu.VMEM((CHUNK, CW), jnp.float32),
            pltpu.VMEM((CHUNK, DIM), jnp.float32),
            pltpu.VMEM((CHUNK, DIM), jnp.float32),
            pltpu.VMEM((CHUNK, DIM), jnp.float32),
            pltpu.SemaphoreType.DMA,
            pltpu.SemaphoreType.DMA,
            pltpu.SemaphoreType.DMA,
            pltpu.SemaphoreType.DMA,
            pltpu.SemaphoreType.DMA,
            pltpu.SemaphoreType.DMA,
            pltpu.SemaphoreType.DMA,
            pltpu.SemaphoreType.DMA,
            pltpu.SemaphoreType.DMA,
        ],
    )(_sc_kernel)
    return f(word_flat, cidx_flat, W, Cpad)


def kernel(word_ids, mask_ids, W, P, Sg):
    word_flat = word_ids.reshape(-1).astype(jnp.int32)
    mask_flat = mask_ids.reshape(-1).astype(jnp.int32)
    pos_flat = jnp.broadcast_to(jnp.arange(S, dtype=jnp.int32),
                                (B, S)).reshape(-1)
    cidx_flat = mask_flat * S + pos_flat
    C = (Sg[:, None, :] + P[None, :S, :]).reshape(SEG * S, DIM)
    Cpad = jnp.pad(C, ((0, 0), (0, CW - DIM)))
    out = _run(word_flat, cidx_flat, W, Cpad)
    return out.reshape(B, S, DIM)
